# baseline (speedup 1.0000x reference)
import jax
import jax.numpy as jnp
from jax import lax
from jax.experimental import pallas as pl
from jax.experimental.pallas import tpu as pltpu
from jax.experimental.pallas import tpu_sc as plsc

BATCH = 4
SEQ_LEN = 8192
D_MODEL = 1024
FLAT = BATCH * SEQ_LEN

NUM_CORES = 2
NUM_SUBCORES = 16
NW = NUM_CORES * NUM_SUBCORES
ROWS_PER_W = FLAT // NW
CHUNK = 32
NCHUNK = ROWS_PER_W // CHUNK


GROUPS = CHUNK * D_MODEL // 16


def _body(ids_hbm, tok_hbm, pos_hbm, out_hbm, idx_v, buf, pbuf):
    wid = lax.axis_index("s") * NUM_CORES + lax.axis_index("c")
    base = wid * ROWS_PER_W
    s0 = lax.rem(base, SEQ_LEN)

    pltpu.sync_copy(ids_hbm.at[wid], idx_v)

    def step(c, carry):
        row0 = base + c * CHUNK
        p0 = s0 + c * CHUNK
        pltpu.sync_copy(pos_hbm.at[pl.ds(p0, CHUNK)], pbuf)
        pltpu.sync_copy(tok_hbm.at[idx_v.at[c]], buf)

        @plsc.parallel_loop(0, GROUPS, unroll=8)
        def add_group(i):
            r = lax.shift_right_logical(i, 6)
            off = lax.mul(lax.rem(i, 64), 16)
            plsc.addupdate(buf.at[r, pl.ds(off, 16)], pbuf[r, pl.ds(off, 16)])

        pltpu.sync_copy(buf, out_hbm.at[pl.ds(row0, CHUNK)])
        return carry

    lax.fori_loop(0, NCHUNK, step, 0)


@jax.jit
def _embed(ids_flat, token_table, pos_table):
    mesh = plsc.VectorSubcoreMesh(core_axis_name="c", subcore_axis_name="s")
    k = pl.kernel(
        _body,
        out_type=jax.ShapeDtypeStruct((FLAT, D_MODEL), jnp.float32),
        mesh=mesh,
        scratch_types=[
            pltpu.VMEM((NCHUNK, CHUNK), jnp.int32),
            pltpu.VMEM((CHUNK, D_MODEL), jnp.float32),
            pltpu.VMEM((CHUNK, D_MODEL), jnp.float32),
        ],
    )
    return k(ids_flat, token_table, pos_table)


def kernel(input_ids, token_table, pos_table):
    ids = input_ids.astype(jnp.int32).reshape(NW, NCHUNK, CHUNK)
    out = _embed(ids, token_table, pos_table)
    return out.reshape(BATCH, SEQ_LEN, D_MODEL)


# speedup vs baseline: 1.5923x; 1.5923x over previous
import jax
import jax.numpy as jnp
from jax import lax
from jax.experimental import pallas as pl
from jax.experimental.pallas import tpu as pltpu
from jax.experimental.pallas import tpu_sc as plsc

BATCH = 4
SEQ_LEN = 8192
D_MODEL = 1024
FLAT = BATCH * SEQ_LEN

NUM_CORES = 2
NUM_SUBCORES = 16
NW = NUM_CORES * NUM_SUBCORES
S_PER_W = SEQ_LEN // NW
CHUNK = 32
NCHUNK_S = S_PER_W // CHUNK
UNITS = NCHUNK_S * BATCH
GROUPS = CHUNK * D_MODEL // 16


def _body(ids_hbm, tok_hbm, pos_hbm, out_hbm,
          idx_v, buf0, buf1, pbuf, gsem0, gsem1, wsem0, wsem1):
    wid = lax.axis_index("s") * NUM_CORES + lax.axis_index("c")
    s_base = wid * S_PER_W

    pltpu.sync_copy(ids_hbm.at[wid], idx_v)

    bufs = (buf0, buf1)
    gsems = (gsem0, gsem1)
    wsems = (wsem0, wsem1)

    def start_gather(u):
        c, b = divmod(u, BATCH)
        return pltpu.async_copy(tok_hbm.at[idx_v.at[c, b]], bufs[u % 2],
                                gsems[u % 2])

    pltpu.sync_copy(pos_hbm.at[pl.ds(s_base, CHUNK)], pbuf)
    gdesc = [start_gather(0), None]
    wdesc = [None, None]

    for u in range(UNITS):
        cur = bufs[u % 2]
        if u + 1 < UNITS:
            nxt = (u + 1) % 2
            if wdesc[nxt] is not None:
                wdesc[nxt].wait()
                wdesc[nxt] = None
            gdesc[nxt] = start_gather(u + 1)
        gdesc[u % 2].wait()

        c, b = divmod(u, BATCH)
        if b == 0 and u > 0:
            pltpu.sync_copy(pos_hbm.at[pl.ds(s_base + c * CHUNK, CHUNK)], pbuf)

        @plsc.parallel_loop(0, GROUPS, unroll=8)
        def add_group(i):
            r = lax.shift_right_logical(i, 6)
            off = lax.mul(lax.rem(i, 64), 16)
            plsc.addupdate(cur.at[r, pl.ds(off, 16)], pbuf[r, pl.ds(off, 16)])

        row0 = b * SEQ_LEN + s_base + c * CHUNK
        wdesc[u % 2] = pltpu.async_copy(cur, out_hbm.at[pl.ds(row0, CHUNK)],
                                        wsems[u % 2])

    for d in wdesc:
        if d is not None:
            d.wait()


@jax.jit
def _embed(ids_r, token_table, pos_table):
    mesh = plsc.VectorSubcoreMesh(core_axis_name="c", subcore_axis_name="s")
    k = pl.kernel(
        _body,
        out_type=jax.ShapeDtypeStruct((FLAT, D_MODEL), jnp.float32),
        mesh=mesh,
        scratch_types=[
            pltpu.VMEM((NCHUNK_S, BATCH, CHUNK), jnp.int32),
            pltpu.VMEM((CHUNK, D_MODEL), jnp.float32),
            pltpu.VMEM((CHUNK, D_MODEL), jnp.float32),
            pltpu.VMEM((CHUNK, D_MODEL), jnp.float32),
            pltpu.SemaphoreType.DMA,
            pltpu.SemaphoreType.DMA,
            pltpu.SemaphoreType.DMA,
            pltpu.SemaphoreType.DMA,
        ],
    )
    return k(ids_r, token_table, pos_table)


def kernel(input_ids, token_table, pos_table):
    ids_r = (input_ids.astype(jnp.int32)
             .reshape(BATCH, NW, NCHUNK_S, CHUNK)
             .transpose(1, 2, 0, 3))
    out = _embed(ids_r, token_table, pos_table)
    return out.reshape(BATCH, SEQ_LEN, D_MODEL)


# speedup vs baseline: 1.8254x; 1.1464x over previous
import jax
import jax.numpy as jnp
from jax import lax
from jax.experimental import pallas as pl
from jax.experimental.pallas import tpu as pltpu
from jax.experimental.pallas import tpu_sc as plsc

BATCH = 4
SEQ_LEN = 8192
D_MODEL = 1024
FLAT = BATCH * SEQ_LEN

NUM_CORES = 2
NUM_SUBCORES = 16
NW = NUM_CORES * NUM_SUBCORES
S_PER_W = SEQ_LEN // NW
CHUNK = 16
NCHUNK_S = S_PER_W // CHUNK
UNITS = NCHUNK_S * BATCH
GROUPS = CHUNK * D_MODEL // 16
G_PER_ROW = D_MODEL // 16
NBUF = 4


def _body(ids_hbm, tok_hbm, pos_hbm, out_hbm, idx_v, *scratch):
    bufs = scratch[:NBUF]
    pbufs = scratch[NBUF:NBUF + 2]
    gsems = scratch[NBUF + 2:2 * NBUF + 2]
    wsems = scratch[2 * NBUF + 2:3 * NBUF + 2]
    psems = scratch[3 * NBUF + 2:]

    wid = lax.axis_index("s") * NUM_CORES + lax.axis_index("c")
    s_base = wid * S_PER_W

    pltpu.sync_copy(ids_hbm.at[wid], idx_v)

    def start_gather(u):
        c, b = divmod(u, BATCH)
        return pltpu.async_copy(tok_hbm.at[idx_v.at[c, b]], bufs[u % NBUF],
                                gsems[u % NBUF])

    def start_pos(c):
        return pltpu.async_copy(pos_hbm.at[pl.ds(s_base + c * CHUNK, CHUNK)],
                                pbufs[c % 2], psems[c % 2])

    pdesc = [start_pos(0), start_pos(1)]
    gdesc = [None] * NBUF
    wdesc = [None] * NBUF
    for u in range(NBUF - 1):
        gdesc[u] = start_gather(u)

    for u in range(UNITS):
        slot = u % NBUF
        nu = u + NBUF - 1
        if nu < UNITS:
            ns = nu % NBUF
            if wdesc[ns] is not None:
                wdesc[ns].wait()
                wdesc[ns] = None
            gdesc[ns] = start_gather(nu)

        c, b = divmod(u, BATCH)
        if b == 0:
            pdesc[c % 2].wait()
            pdesc[c % 2] = None
        gdesc[slot].wait()

        cur = bufs[slot]
        pb = pbufs[c % 2]

        @plsc.parallel_loop(0, GROUPS, unroll=8)
        def add_group(i):
            r = lax.div(i, G_PER_ROW)
            off = lax.mul(lax.rem(i, G_PER_ROW), 16)
            plsc.addupdate(cur.at[r, pl.ds(off, 16)], pb[r, pl.ds(off, 16)])

        if b == BATCH - 1 and c + 2 < NCHUNK_S:
            pdesc[c % 2] = start_pos(c + 2)

        row0 = b * SEQ_LEN + s_base + c * CHUNK
        wdesc[slot] = pltpu.async_copy(cur, out_hbm.at[pl.ds(row0, CHUNK)],
                                       wsems[slot])

    for d in wdesc:
        if d is not None:
            d.wait()


@jax.jit
def _embed(ids_r, token_table, pos_table):
    mesh = plsc.VectorSubcoreMesh(core_axis_name="c", subcore_axis_name="s")
    k = pl.kernel(
        _body,
        out_type=jax.ShapeDtypeStruct((FLAT, D_MODEL), jnp.float32),
        mesh=mesh,
        scratch_types=(
            [pltpu.VMEM((NCHUNK_S, BATCH, CHUNK), jnp.int32)]
            + [pltpu.VMEM((CHUNK, D_MODEL), jnp.float32)] * NBUF
            + [pltpu.VMEM((CHUNK, D_MODEL), jnp.float32)] * 2
            + [pltpu.SemaphoreType.DMA] * (2 * NBUF + 2)
        ),
    )
    return k(ids_r, token_table, pos_table)


def kernel(input_ids, token_table, pos_table):
    ids_r = (input_ids.astype(jnp.int32)
             .reshape(BATCH, NW, NCHUNK_S, CHUNK)
             .transpose(1, 2, 0, 3))
    out = _embed(ids_r, token_table, pos_table)
    return out.reshape(BATCH, SEQ_LEN, D_MODEL)
